# baseline (speedup 1.0000x reference)
import dataclasses
import functools

import jax
import jax.numpy as jnp
from jax import lax
from jax.experimental import pallas as pl
from jax.experimental.pallas import tpu as pltpu
from jax.experimental.pallas import tpu_sc as plsc

NC = 2
NS = 16
NW = NC * NS
CH = 64


def _sc_cp():
    cp = pltpu.CompilerParams()
    if "needs_layout_passes" in pltpu.CompilerParams.__dataclass_fields__:
        cp = dataclasses.replace(cp, needs_layout_passes=False)
    return cp


def _mesh():
    return plsc.VectorSubcoreMesh(core_axis_name="c", subcore_axis_name="s",
                                  num_cores=NC, num_subcores=NS)


def _wid():
    return lax.axis_index("s") * NC + lax.axis_index("c")


def _pad_rows(a, mult):
    n = a.shape[0]
    rem = (-n) % mult
    if rem:
        a = jnp.pad(a, ((0, rem),) + ((0, 0),) * (a.ndim - 1))
    return a


def _mm_kernel(x_ref, w_ref, b_ref, o_ref, *, act):
    acc = jnp.dot(x_ref[...], w_ref[...], preferred_element_type=jnp.float32)
    acc = acc + b_ref[...]
    if act == "leaky":
        acc = jnp.where(acc >= 0, acc, 0.01 * acc)
    o_ref[...] = acc


def _dense(x, w, b, act=None, block=512, out_rows=None):
    n = x.shape[0] if out_rows is None else out_rows
    xp = _pad_rows(x, block)
    npad = xp.shape[0]
    k, dout = w.shape
    out = pl.pallas_call(
        functools.partial(_mm_kernel, act=act),
        grid=(npad // block,),
        in_specs=[
            pl.BlockSpec((block, k), lambda i: (i, 0)),
            pl.BlockSpec((k, dout), lambda i: (0, 0)),
            pl.BlockSpec((dout,), lambda i: (0,)),
        ],
        out_specs=pl.BlockSpec((block, dout), lambda i: (i, 0)),
        out_shape=jax.ShapeDtypeStruct((npad, dout), jnp.float32),
    )(xp, w, b)
    return out[:n]


def _k0_permute_ea(ea, perm, epad):
    e_per_w = epad // NW
    nch = e_per_w // CH
    de = ea.shape[1]

    @functools.partial(
        pl.kernel, mesh=_mesh(), compiler_params=_sc_cp(),
        out_type=jax.ShapeDtypeStruct((epad, de), jnp.float32),
        scratch_types=[pltpu.VMEM((CH,), jnp.int32),
                       pltpu.VMEM((CH, de), jnp.float32),
                       pltpu.SemaphoreType.DMA])
    def k(ea_hbm, perm_hbm, out_hbm, idx_v, rows_v, sem):
        tbase = _wid() * e_per_w

        @pl.loop(0, nch)
        def _(c):
            base = tbase + c * CH
            pltpu.sync_copy(perm_hbm.at[pl.ds(base, CH)], idx_v)
            pltpu.async_copy(ea_hbm.at[idx_v], rows_v, sem).wait()
            pltpu.sync_copy(rows_v, out_hbm.at[pl.ds(base, CH)])

    return k(ea, perm)


def _k1_alpha(qcat, kv, eap, dsts, srcs, epad, dout, dq):
    e_per_w = epad // NW
    nch = e_per_w // CH
    nk = dout // 16
    ne = 8
    inv = 1.0 / (dout ** 0.5)

    @functools.partial(
        pl.kernel, mesh=_mesh(), compiler_params=_sc_cp(),
        out_type=[jax.ShapeDtypeStruct((epad,), jnp.float32),
                  jax.ShapeDtypeStruct((epad, dout), jnp.float32)],
        scratch_types=[pltpu.VMEM((CH,), jnp.int32),
                       pltpu.VMEM((CH,), jnp.int32),
                       pltpu.VMEM((CH, dq), jnp.float32),
                       pltpu.VMEM((CH, 2 * dout), jnp.float32),
                       pltpu.VMEM((CH, 128), jnp.float32),
                       pltpu.VMEM((CH,), jnp.float32),
                       pltpu.SemaphoreType.DMA])
    def k(qcat_hbm, kv_hbm, eap_hbm, dst_hbm, src_hbm,
          alpha_hbm, vs_hbm, dstc, srcc, qrows, kvrows, earows, abuf, sem):
        tbase = _wid() * e_per_w

        @pl.loop(0, nch)
        def _(c):
            base = tbase + c * CH
            pltpu.sync_copy(dst_hbm.at[pl.ds(base, CH)], dstc)
            pltpu.sync_copy(src_hbm.at[pl.ds(base, CH)], srcc)
            cp1 = pltpu.async_copy(qcat_hbm.at[dstc], qrows, sem)
            cp2 = pltpu.async_copy(kv_hbm.at[srcc], kvrows, sem)
            pltpu.sync_copy(eap_hbm.at[pl.ds(base, CH)], earows)
            cp1.wait()
            cp2.wait()

            @pl.loop(0, CH // 16)
            def _(g):
                avec = jnp.zeros((16,), jnp.float32)
                for j in range(16):
                    e = g * 16 + j
                    acc = qrows[e, pl.ds(0, 16)] * kvrows[e, pl.ds(0, 16)]
                    for t in range(1, nk):
                        acc += (qrows[e, pl.ds(16 * t, 16)]
                                * kvrows[e, pl.ds(16 * t, 16)])
                    for t in range(ne):
                        acc += (qrows[e, pl.ds(dout + 16 * t, 16)]
                                * earows[e, pl.ds(16 * t, 16)])
                    a = jnp.sum(acc) * inv
                    avec = jnp.where(lax.iota(jnp.int32, 16) == j, a, avec)
                abuf[pl.ds(g * 16, 16)] = avec

            pltpu.sync_copy(abuf, alpha_hbm.at[pl.ds(base, CH)])
            pltpu.sync_copy(kvrows.at[:, pl.ds(dout, dout)],
                            vs_hbm.at[pl.ds(base, CH)])

    return k(qcat, kv, eap, dsts, srcs)


def _k2_segsum(alpha, dstl, vs, eap, epad, n_nodes, dout, dq):
    e_per_w = epad // NW
    nch = e_per_w // CH
    nk = dout // 16
    ne = 8
    npc = nk + ne
    row_w = dq + 128
    neg = jnp.float32(-1e30)

    @functools.partial(
        pl.kernel, mesh=_mesh(), compiler_params=_sc_cp(),
        out_type=[jax.ShapeDtypeStruct((n_nodes + 8, row_w), jnp.float32),
                  jax.ShapeDtypeStruct((2 * NW, 16), jnp.float32),
                  jax.ShapeDtypeStruct((2 * NW, row_w), jnp.float32)],
        scratch_types=[pltpu.VMEM((CH,), jnp.float32),
                       pltpu.VMEM((CH + 16,), jnp.int32),
                       pltpu.VMEM((CH, dout), jnp.float32),
                       pltpu.VMEM((CH, 128), jnp.float32),
                       pltpu.VMEM((CH, row_w), jnp.float32),
                       pltpu.VMEM((CH,), jnp.int32),
                       pltpu.VMEM((2 * row_w,), jnp.float32),
                       pltpu.VMEM((16,), jnp.float32),
                       pltpu.SemaphoreType.DMA])
    def k(alpha_hbm, dstl_hbm, vs_hbm, eap_hbm,
          agg_hbm, fixmeta_hbm, fixrow_hbm,
          abuf, dbuf, vbuf, ebuf, stag2d, idbuf, fbuf, mbuf, sem):
        w = _wid()
        tbase = w * e_per_w
        iota = lax.iota(jnp.int32, 16)
        dumpid = jnp.int32(n_nodes)

        def group(g, carry):
            m, den, fd, meta0, *accs = carry
            accs = list(accs)
            a16 = abuf[pl.ds(g * 16, 16)]
            d16 = dbuf[pl.ds(g * 16, 16)]
            n16 = dbuf[pl.ds(g * 16 + 1, 16)]
            idvec = jnp.full((16,), dumpid, jnp.int32)
            for j in range(16):
                a = a16[j]
                mn = jnp.maximum(m, a)
                scv = jnp.exp(jnp.full((16,), m - mn, jnp.float32))
                wv = jnp.exp(jnp.full((16,), a - mn, jnp.float32))
                e = g * 16 + j
                for t in range(nk):
                    accs[t] = accs[t] * scv + wv * vbuf[e, pl.ds(16 * t, 16)]
                for t in range(ne):
                    accs[nk + t] = (accs[nk + t] * scv
                                    + wv * ebuf[e, pl.ds(16 * t, 16)])
                den = den * scv[0] + wv[0]
                m = mn
                node = d16[j]
                isend = node != n16[j]
                idvec = jnp.where(
                    jnp.logical_and(iota == j, jnp.logical_and(isend, fd > 0)),
                    node, idvec)

                def store_rows():
                    for t in range(npc):
                        stag2d[e, pl.ds(16 * t, 16)] = accs[t]
                    stag2d[e, pl.ds(dq, 16)] = jnp.full(
                        (16,), den, jnp.float32)

                def on_end(meta0_):
                    def first(mm):
                        for t in range(npc):
                            fbuf[pl.ds(16 * t, 16)] = accs[t]
                        mv = jnp.where(iota == 0, node.astype(jnp.float32), 0.0)
                        mv = jnp.where(iota == 1, m, mv)
                        mv = jnp.where(iota == 2, den, mv)
                        return mv

                    def interior(mm):
                        store_rows()
                        return mm
                    return lax.cond(fd > 0, interior, first, meta0_)

                meta0 = lax.cond(isend, on_end, lambda mm: mm, meta0)
                rs = jnp.where(isend, jnp.float32(0.0), jnp.float32(1.0))
                rsv = jnp.full((16,), rs, jnp.float32)
                for t in range(npc):
                    accs[t] = accs[t] * rsv
                den = den * rs
                m = jnp.where(isend, neg, m)
                fd = jnp.where(isend, jnp.int32(1), fd)
            idbuf[pl.ds(g * 16, 16)] = idvec
            return (m, den, fd, meta0, *accs)

        def chunk(c, carry):
            base = tbase + c * CH
            pltpu.sync_copy(alpha_hbm.at[pl.ds(base, CH)], abuf)
            pltpu.sync_copy(dstl_hbm.at[pl.ds(base, CH + 16)], dbuf)
            pltpu.sync_copy(vs_hbm.at[pl.ds(base, CH)], vbuf)
            pltpu.sync_copy(eap_hbm.at[pl.ds(base, CH)], ebuf)
            carry = lax.fori_loop(0, CH // 16, group, carry)
            pltpu.async_copy(stag2d, agg_hbm.at[idbuf], sem).wait()
            return carry

        zero = jnp.zeros((16,), jnp.float32)
        meta_init = jnp.where(iota == 0, jnp.float32(-1.0), jnp.float32(0.0))
        init = (neg, jnp.float32(0.0), jnp.int32(0), meta_init,
                *([zero] * npc))
        m, den, fd, meta0, *accs = lax.fori_loop(0, nch, chunk, init)

        last = dbuf[pl.ds(CH - 16, 16)]
        has_last = den > 0
        meta1 = jnp.where(iota == 0,
                          jnp.where(has_last, last[15].astype(jnp.float32),
                                    jnp.float32(-1.0)), 0.0)
        meta1 = jnp.where(iota == 1, m, meta1)
        meta1 = jnp.where(iota == 2, den, meta1)
        for t in range(npc):
            fbuf[pl.ds(row_w + 16 * t, 16)] = accs[t]
        mbuf[...] = meta0
        pltpu.sync_copy(mbuf, fixmeta_hbm.at[2 * w])
        mbuf[...] = meta1
        pltpu.sync_copy(mbuf, fixmeta_hbm.at[2 * w + 1])
        pltpu.sync_copy(fbuf.at[pl.ds(0, row_w)], fixrow_hbm.at[2 * w])
        pltpu.sync_copy(fbuf.at[pl.ds(row_w, row_w)], fixrow_hbm.at[2 * w + 1])

    return k(alpha, dstl, vs, eap)


def _combine(aggflat, fixmeta, fixrow, deg, n_nodes, dq):
    row_w = dq + 128
    rows = aggflat.reshape(n_nodes + 8, row_w)[:n_nodes]
    den_d = rows[:, dq]
    agg_d = rows[:, :dq] / jnp.maximum(den_d, 1e-30)[:, None]

    fid = fixmeta[:, 0]
    fm = fixmeta[:, 1]
    fden = fixmeta[:, 2]
    valid = fid >= 0
    ids = jnp.where(valid, fid, n_nodes).astype(jnp.int32)
    gmax = jnp.full((n_nodes + 1,), -1e30, jnp.float32).at[ids].max(
        jnp.where(valid, fm, -1e30))
    sc = jnp.where(valid, jnp.exp(fm - gmax[ids]), 0.0)
    gden = jnp.zeros((n_nodes + 1,), jnp.float32).at[ids].add(fden * sc)
    grow = jnp.zeros((n_nodes + 1, dq), jnp.float32).at[ids].add(
        fixrow[:, :dq] * sc[:, None])
    bmask = jnp.zeros((n_nodes + 1,), jnp.bool_).at[ids].max(valid)
    agg = jnp.where(bmask[:n_nodes, None],
                    grow[:n_nodes] / jnp.maximum(gden[:n_nodes], 1e-30)[:, None],
                    agg_d)
    return jnp.where((deg > 0)[:, None], agg, 0.0)


def _layer(x, eap, dsts, srcs, dstl, deg, epad, n_nodes,
           Wq, bq, Wk, bk, Wv, bv, Weg, Ws, bs):
    dout = Wq.shape[1]
    dq = dout + 128
    Wqcat = jnp.concatenate([Wq, Wq @ Weg.T], axis=1)
    bqcat = jnp.concatenate([bq, bq @ Weg.T], axis=0)
    Wkv = jnp.concatenate([Wk, Wv], axis=1)
    bkv = jnp.concatenate([bk, bv], axis=0)
    qcat = _dense(x, Wqcat, bqcat, out_rows=None)
    kv = _dense(x, Wkv, bkv, out_rows=None)
    qcat = jnp.pad(qcat, ((0, 8), (0, 0)))
    kv = jnp.pad(kv, ((0, 8), (0, 0)))

    alpha, vs = _k1_alpha(qcat, kv, eap, dsts, srcs, epad, dout, dq)
    aggflat, fixmeta, fixrow = _k2_segsum(alpha, dstl, vs, eap, epad,
                                          n_nodes, dout, dq)
    agg = _combine(aggflat, fixmeta, fixrow, deg, n_nodes, dq)
    aggv = agg[:, :dout]
    agge = agg[:, dout:]
    out = (aggv
           + _dense(agge, Weg, jnp.zeros((dout,), jnp.float32))
           + _dense(x, Ws, bs))
    return jnp.where(out >= 0, out, 0.01 * out)


def kernel(seq, node_s, edge_index, edge_s, batch, embed, Wn, bn, Wep, bep,
           Wq0, bq0, Wk0, bk0, Wv0, bv0, Weg0, Ws0, bs0,
           Wq1, bq1, Wk1, bk1, Wv1, bv1, Weg1, Ws1, bs1,
           Wq2, bq2, Wk2, bk2, Wv2, bv2, Weg2, Ws2, bs2,
           Wout, bout):
    n_nodes = seq.shape[0]
    e_edges = edge_index.shape[1]
    src = edge_index[0].astype(jnp.int32)
    dst = edge_index[1].astype(jnp.int32)

    eidx = jnp.arange(e_edges, dtype=jnp.int32)
    dsts, srcs, perm = lax.sort((dst, src, eidx), num_keys=1)
    epad = -(-e_edges // (NW * CH)) * (NW * CH)
    padn = epad - e_edges
    dsts = jnp.concatenate([dsts, jnp.full((padn,), n_nodes, jnp.int32)])
    srcs = jnp.concatenate([srcs, jnp.zeros((padn,), jnp.int32)])
    perm = jnp.concatenate([perm, jnp.zeros((padn,), jnp.int32)])
    dstl = jnp.concatenate([dsts, jnp.full((16,), n_nodes, jnp.int32)])
    deg = (jnp.searchsorted(dsts, jnp.arange(1, n_nodes + 1, dtype=jnp.int32))
           - jnp.searchsorted(dsts, jnp.arange(n_nodes, dtype=jnp.int32)))

    x = jnp.concatenate([embed[seq], node_s], axis=-1)
    x = _dense(x, Wn, bn)
    ea = _dense(edge_s, Wep, bep)
    ea = jnp.pad(ea, ((0, 8), (0, 0)))
    eap = _k0_permute_ea(ea, perm, epad)

    args = (eap, dsts, srcs, dstl, deg, epad, n_nodes)
    x = _layer(x, *args, Wq0, bq0, Wk0, bk0, Wv0, bv0, Weg0, Ws0, bs0)
    x = _layer(x, *args, Wq1, bq1, Wk1, bk1, Wv1, bv1, Weg1, Ws1, bs1)
    x = _layer(x, *args, Wq2, bq2, Wk2, bk2, Wv2, bv2, Weg2, Ws2, bs2)

    b32 = batch.astype(jnp.int32)
    cnt = jax.ops.segment_sum(jnp.ones((n_nodes,), jnp.float32), b32,
                              num_segments=32)
    pooled = (jax.ops.segment_sum(x, b32, num_segments=32)
              / jnp.maximum(cnt, 1.0)[:, None])
    return _dense(pooled, Wout, bout)
